# baseline (speedup 1.0000x reference)
import dataclasses
import functools

import jax
import jax.numpy as jnp
from jax import lax
from jax.experimental import pallas as pl
from jax.experimental.pallas import tpu as pltpu
from jax.experimental.pallas import tpu_sc as plsc

D = 128
L = 16
NC, NS = 2, 16
NW = NC * NS
C = 80
B = 400


def _normalize_body(z_ref, zn_ref):
    z = z_ref[...]
    norm = jnp.sqrt(jnp.sum(z * z, axis=1, keepdims=True))
    zn_ref[...] = (z / jnp.maximum(norm, 1e-12)).astype(jnp.bfloat16)


def _sc_body(zn_i32_hbm, src_hbm, dst_hbm, out_hbm,
             table, idx_src, idx_dst, s0, d0, s1, d1, scores, accbuf,
             sem0, sem1):
    n = zn_i32_hbm.shape[0]
    per_w = out_hbm.shape[0] // NW
    cid = lax.axis_index("c")
    sid = lax.axis_index("s")
    wid = sid * NC + cid
    base = wid * per_w
    lane = lax.iota(jnp.int32, L)

    slab = (n // NS) // 8 * 8
    tail = n - (NS - 1) * slab

    @pl.when(sid < NS - 1)
    def _stage_body():
        pltpu.sync_copy(zn_i32_hbm.at[pl.ds(sid * slab, slab)],
                        table.at[pl.ds(sid * slab, slab)])

    @pl.when(sid == NS - 1)
    def _stage_tail():
        pltpu.sync_copy(zn_i32_hbm.at[pl.ds((NS - 1) * slab, tail)],
                        table.at[pl.ds((NS - 1) * slab, tail)])
    plsc.subcore_barrier()

    def issue(ci, sbuf, dbuf, sem):
        o = ci * C
        pltpu.async_copy(table.at[idx_src.at[pl.ds(o, C)]], sbuf, sem)
        pltpu.async_copy(table.at[idx_dst.at[pl.ds(o, C)]], dbuf, sem)

    def wait(ci_buf, sem):
        sbuf, dbuf = ci_buf
        pltpu.make_async_copy(zn_i32_hbm.at[pl.ds(0, C)], sbuf, sem).wait()
        pltpu.make_async_copy(zn_i32_hbm.at[pl.ds(0, C)], dbuf, sem).wait()

    def compute(ci, sbuf, dbuf):
        @pl.loop(0, C // L)
        def _group(g):
            for j in range(L):
                e = g * L + j
                acc = None
                for c in range(D // (2 * L)):
                    a = plsc.bitcast(sbuf[e, pl.ds(c * L, L)], jnp.bfloat16)
                    b = plsc.bitcast(dbuf[e, pl.ds(c * L, L)], jnp.bfloat16)
                    lo, hi = plsc.unpack(a * b, format=plsc.PackFormat.INTERLEAVED)
                    part = lo + hi
                    acc = part if acc is None else acc + part
                accbuf[j, pl.ds(0, L)] = acc
            cols = [
                plsc.load_gather(
                    accbuf, [lane, jnp.full((L,), c, jnp.int32)])
                for c in range(L)
            ]
            while len(cols) > 1:
                cols = [a + b for a, b in zip(cols[::2], cols[1::2])]
            scores[pl.ds(ci * C + g * L, L)] = jnp.exp(2.0 * cols[0] - 2.0)

    nch = B // C

    @pl.loop(0, per_w // B)
    def _block(blk):
        bbase = base + blk * B
        pltpu.sync_copy(src_hbm.at[pl.ds(bbase, B)], idx_src)
        pltpu.sync_copy(dst_hbm.at[pl.ds(bbase, B)], idx_dst)

        issue(0, s0, d0, sem0)

        @pl.loop(0, nch // 2)
        def _pair(it):
            i = 2 * it
            issue(i + 1, s1, d1, sem1)
            wait((s0, d0), sem0)
            compute(i, s0, d0)
            issue(i + 2, s0, d0, sem0)
            wait((s1, d1), sem1)
            compute(i + 1, s1, d1)

        wait((s0, d0), sem0)
        compute(nch - 1, s0, d0)

        pltpu.sync_copy(scores, out_hbm.at[pl.ds(bbase, B)])


def kernel(z, edge_index):
    n, d = z.shape
    e = edge_index.shape[1]
    assert d == D and e % (NW * B) == 0 and (B // C) % 2 == 1

    zn = pl.pallas_call(
        _normalize_body,
        out_shape=jax.ShapeDtypeStruct((n, d), jnp.bfloat16),
    )(z)
    zn_half = jax.lax.bitcast_convert_type(
        zn.reshape(n, d // 2, 2), jnp.int32)
    zn_i32 = jnp.concatenate([zn_half, zn_half], axis=1)

    idx = edge_index.astype(jnp.int32)
    src = idx[0]
    dst = idx[1]

    cp = pltpu.CompilerParams()
    if "needs_layout_passes" in pltpu.CompilerParams.__dataclass_fields__:
        cp = dataclasses.replace(cp, needs_layout_passes=False)

    mesh = plsc.VectorSubcoreMesh(core_axis_name="c", subcore_axis_name="s")
    sc_kernel = pl.kernel(
        _sc_body,
        out_type=jax.ShapeDtypeStruct((e,), jnp.float32),
        mesh=mesh,
        scratch_types=[
            pltpu.VMEM_SHARED((n, D), jnp.int32),
            pltpu.VMEM((B,), jnp.int32),
            pltpu.VMEM((B,), jnp.int32),
            pltpu.VMEM((C, D), jnp.int32),
            pltpu.VMEM((C, D), jnp.int32),
            pltpu.VMEM((C, D), jnp.int32),
            pltpu.VMEM((C, D), jnp.int32),
            pltpu.VMEM((B,), jnp.float32),
            pltpu.VMEM((L, L), jnp.float32),
            pltpu.SemaphoreType.DMA,
            pltpu.SemaphoreType.DMA,
        ],
        compiler_params=cp,
    )
    return sc_kernel(zn_i32, src, dst)


# speedup vs baseline: 1.0183x; 1.0183x over previous
import dataclasses
import functools

import jax
import jax.numpy as jnp
from jax import lax
from jax.experimental import pallas as pl
from jax.experimental.pallas import tpu as pltpu
from jax.experimental.pallas import tpu_sc as plsc

D = 128
L = 16
NC, NS = 2, 16
NW = NC * NS
C = 80
B = 400


def _normalize_body(z_ref, zn_ref):
    z = z_ref[...]
    norm = jnp.sqrt(jnp.sum(z * z, axis=1, keepdims=True))
    zn_ref[...] = (z / jnp.maximum(norm, 1e-12)).astype(jnp.bfloat16)


def _sc_body(zn_i32_hbm, src_hbm, dst_hbm, out_hbm,
             table, idx_src, idx_dst, s0, d0, s1, d1, scores, accbuf,
             sem0, sem1):
    n = zn_i32_hbm.shape[0]
    per_w = out_hbm.shape[0] // NW
    cid = lax.axis_index("c")
    sid = lax.axis_index("s")
    wid = sid * NC + cid
    base = wid * per_w
    lane = lax.iota(jnp.int32, L)

    slab = (n // NS) // 8 * 8
    tail = n - (NS - 1) * slab

    @pl.when(sid < NS - 1)
    def _stage_body():
        pltpu.sync_copy(zn_i32_hbm.at[pl.ds(sid * slab, slab)],
                        table.at[pl.ds(sid * slab, slab)])

    @pl.when(sid == NS - 1)
    def _stage_tail():
        pltpu.sync_copy(zn_i32_hbm.at[pl.ds((NS - 1) * slab, tail)],
                        table.at[pl.ds((NS - 1) * slab, tail)])
    plsc.subcore_barrier()

    def issue(ci, sbuf, dbuf, sem):
        o = ci * C
        pltpu.async_copy(table.at[idx_src.at[pl.ds(o, C)]], sbuf, sem)
        pltpu.async_copy(table.at[idx_dst.at[pl.ds(o, C)]], dbuf, sem)

    def wait(ci_buf, sem):
        sbuf, dbuf = ci_buf
        pltpu.make_async_copy(zn_i32_hbm.at[pl.ds(0, C)], sbuf, sem).wait()
        pltpu.make_async_copy(zn_i32_hbm.at[pl.ds(0, C)], dbuf, sem).wait()

    def compute(ci, sbuf, dbuf):
        @pl.loop(0, C // L)
        def _group(g):
            for j in range(L):
                e = g * L + j
                acc = None
                for c in range(D // (2 * L)):
                    a = plsc.bitcast(sbuf[e, pl.ds(c * L, L)], jnp.bfloat16)
                    b = plsc.bitcast(dbuf[e, pl.ds(c * L, L)], jnp.bfloat16)
                    p = a * b
                    acc = p if acc is None else acc + p
                lo, hi = plsc.unpack(acc, format=plsc.PackFormat.INTERLEAVED)
                accbuf[j, pl.ds(0, L)] = lo + hi
            cols = [
                plsc.load_gather(
                    accbuf, [lane, jnp.full((L,), c, jnp.int32)])
                for c in range(L)
            ]
            while len(cols) > 1:
                cols = [a + b for a, b in zip(cols[::2], cols[1::2])]
            scores[pl.ds(ci * C + g * L, L)] = jnp.exp(2.0 * cols[0] - 2.0)

    nch = B // C

    @pl.loop(0, per_w // B)
    def _block(blk):
        bbase = base + blk * B
        pltpu.sync_copy(src_hbm.at[pl.ds(bbase, B)], idx_src)
        pltpu.sync_copy(dst_hbm.at[pl.ds(bbase, B)], idx_dst)

        issue(0, s0, d0, sem0)

        @pl.loop(0, nch // 2)
        def _pair(it):
            i = 2 * it
            issue(i + 1, s1, d1, sem1)
            wait((s0, d0), sem0)
            compute(i, s0, d0)
            issue(i + 2, s0, d0, sem0)
            wait((s1, d1), sem1)
            compute(i + 1, s1, d1)

        wait((s0, d0), sem0)
        compute(nch - 1, s0, d0)

        pltpu.sync_copy(scores, out_hbm.at[pl.ds(bbase, B)])


def kernel(z, edge_index):
    n, d = z.shape
    e = edge_index.shape[1]
    assert d == D and e % (NW * B) == 0 and (B // C) % 2 == 1

    zn = pl.pallas_call(
        _normalize_body,
        out_shape=jax.ShapeDtypeStruct((n, d), jnp.bfloat16),
    )(z)
    zn_half = jax.lax.bitcast_convert_type(
        zn.reshape(n, d // 2, 2), jnp.int32)
    zn_i32 = jnp.concatenate([zn_half, zn_half], axis=1)

    idx = edge_index.astype(jnp.int32)
    src = idx[0]
    dst = idx[1]

    cp = pltpu.CompilerParams()
    if "needs_layout_passes" in pltpu.CompilerParams.__dataclass_fields__:
        cp = dataclasses.replace(cp, needs_layout_passes=False)

    mesh = plsc.VectorSubcoreMesh(core_axis_name="c", subcore_axis_name="s")
    sc_kernel = pl.kernel(
        _sc_body,
        out_type=jax.ShapeDtypeStruct((e,), jnp.float32),
        mesh=mesh,
        scratch_types=[
            pltpu.VMEM_SHARED((n, D), jnp.int32),
            pltpu.VMEM((B,), jnp.int32),
            pltpu.VMEM((B,), jnp.int32),
            pltpu.VMEM((C, D), jnp.int32),
            pltpu.VMEM((C, D), jnp.int32),
            pltpu.VMEM((C, D), jnp.int32),
            pltpu.VMEM((C, D), jnp.int32),
            pltpu.VMEM((B,), jnp.float32),
            pltpu.VMEM((L, L), jnp.float32),
            pltpu.SemaphoreType.DMA,
            pltpu.SemaphoreType.DMA,
        ],
        compiler_params=cp,
    )
    return sc_kernel(zn_i32, src, dst)


# speedup vs baseline: 1.2025x; 1.1809x over previous
import dataclasses
import functools

import jax
import jax.numpy as jnp
from jax import lax
from jax.experimental import pallas as pl
from jax.experimental.pallas import tpu as pltpu
from jax.experimental.pallas import tpu_sc as plsc

D = 128
L = 16
NC, NS = 2, 16
NW = NC * NS
C = 80
B = 400


def _normalize_body(z_ref, zn_ref):
    z = z_ref[...]
    norm = jnp.sqrt(jnp.sum(z * z, axis=1, keepdims=True))
    zn_ref[...] = (z / jnp.maximum(norm, 1e-12)).astype(jnp.bfloat16)


def _sc_body(zn_bf_hbm, zn_i32_hbm, src_hbm, dst_hbm, out_hbm,
             table, idx_src, idx_dst, s0, d0, s1, d1, scores, accbuf,
             stage_bf, stage_i32, sem0, sem1):
    n = zn_i32_hbm.shape[0]
    per_w = out_hbm.shape[0] // NW
    cid = lax.axis_index("c")
    sid = lax.axis_index("s")
    wid = sid * NC + cid
    base = wid * per_w
    lane = lax.iota(jnp.int32, L)

    sc_rows = stage_bf.shape[0]
    slab = ((n // NS) // 16 + 1) * 16
    start = sid * slab

    @pl.loop(0, slab // sc_rows)
    def _stage(si):
        r0 = start + si * sc_rows

        @pl.when(r0 < n)
        def _do():
            pltpu.sync_copy(zn_bf_hbm.at[pl.ds(r0, sc_rows)], stage_bf)

            @pl.loop(0, sc_rows)
            def _row(r):
                for c in range(D // (2 * L)):
                    v = stage_bf[r, pl.ds(c * 2 * L, 2 * L)]
                    stage_i32[r, pl.ds(c * L, L)] = plsc.bitcast(v, jnp.int32)

            pltpu.sync_copy(stage_i32, table.at[pl.ds(r0, sc_rows)])
    plsc.subcore_barrier()

    def issue(ci, sbuf, dbuf, sem):
        o = ci * C
        pltpu.async_copy(table.at[idx_src.at[pl.ds(o, C)]], sbuf, sem)
        pltpu.async_copy(table.at[idx_dst.at[pl.ds(o, C)]], dbuf, sem)

    def wait(ci_buf, sem):
        sbuf, dbuf = ci_buf
        pltpu.make_async_copy(zn_i32_hbm.at[pl.ds(0, C)], sbuf, sem).wait()
        pltpu.make_async_copy(zn_i32_hbm.at[pl.ds(0, C)], dbuf, sem).wait()

    def compute(ci, sbuf, dbuf):
        @pl.loop(0, C // L)
        def _group(g):
            for j in range(L):
                e = g * L + j
                acc = None
                for c in range(D // (2 * L)):
                    a = plsc.bitcast(sbuf[e, pl.ds(c * L, L)], jnp.bfloat16)
                    b = plsc.bitcast(dbuf[e, pl.ds(c * L, L)], jnp.bfloat16)
                    p = a * b
                    acc = p if acc is None else acc + p
                lo, hi = plsc.unpack(acc, format=plsc.PackFormat.INTERLEAVED)
                accbuf[j, pl.ds(0, L)] = lo + hi
            cols = [
                plsc.load_gather(
                    accbuf, [lane, jnp.full((L,), c, jnp.int32)])
                for c in range(L)
            ]
            while len(cols) > 1:
                cols = [a + b for a, b in zip(cols[::2], cols[1::2])]
            scores[pl.ds(ci * C + g * L, L)] = jnp.exp(2.0 * cols[0] - 2.0)

    nch = B // C

    @pl.loop(0, per_w // B)
    def _block(blk):
        bbase = base + blk * B
        pltpu.sync_copy(src_hbm.at[pl.ds(bbase, B)], idx_src)
        pltpu.sync_copy(dst_hbm.at[pl.ds(bbase, B)], idx_dst)

        issue(0, s0, d0, sem0)

        @pl.loop(0, nch // 2)
        def _pair(it):
            i = 2 * it
            issue(i + 1, s1, d1, sem1)
            wait((s0, d0), sem0)
            compute(i, s0, d0)
            issue(i + 2, s0, d0, sem0)
            wait((s1, d1), sem1)
            compute(i + 1, s1, d1)

        wait((s0, d0), sem0)
        compute(nch - 1, s0, d0)

        pltpu.sync_copy(scores, out_hbm.at[pl.ds(bbase, B)])


def kernel(z, edge_index):
    n, d = z.shape
    e = edge_index.shape[1]
    assert d == D and e % (NW * B) == 0 and (B // C) % 2 == 1

    zn = pl.pallas_call(
        _normalize_body,
        out_shape=jax.ShapeDtypeStruct((n, d), jnp.bfloat16),
    )(z)
    zn_i32 = jax.lax.bitcast_convert_type(
        zn.reshape(n, d // 2, 2), jnp.int32)

    idx = edge_index.astype(jnp.int32)
    src = idx[0]
    dst = idx[1]

    cp = pltpu.CompilerParams(use_tc_tiling_on_sc=False)
    if "needs_layout_passes" in pltpu.CompilerParams.__dataclass_fields__:
        cp = dataclasses.replace(cp, needs_layout_passes=False)

    mesh = plsc.VectorSubcoreMesh(core_axis_name="c", subcore_axis_name="s")
    sc_kernel = pl.kernel(
        _sc_body,
        out_type=jax.ShapeDtypeStruct((e,), jnp.float32),
        mesh=mesh,
        scratch_types=[
            pltpu.VMEM_SHARED((n, D // 2), jnp.int32),
            pltpu.VMEM((B,), jnp.int32),
            pltpu.VMEM((B,), jnp.int32),
            pltpu.VMEM((C, D // 2), jnp.int32),
            pltpu.VMEM((C, D // 2), jnp.int32),
            pltpu.VMEM((C, D // 2), jnp.int32),
            pltpu.VMEM((C, D // 2), jnp.int32),
            pltpu.VMEM((B,), jnp.float32),
            pltpu.VMEM((L, L), jnp.float32),
            pltpu.VMEM((C, D), jnp.bfloat16),
            pltpu.VMEM((C, D // 2), jnp.int32),
            pltpu.SemaphoreType.DMA,
            pltpu.SemaphoreType.DMA,
        ],
        compiler_params=cp,
    )
    return sc_kernel(zn, zn_i32, src, dst)


# speedup vs baseline: 1.2026x; 1.0001x over previous
import dataclasses

import jax
import jax.numpy as jnp
from jax import lax
from jax.experimental import pallas as pl
from jax.experimental.pallas import tpu as pltpu
from jax.experimental.pallas import tpu_sc as plsc

D = 128
L = 16
NC, NS = 2, 16
NW = NC * NS
C = 80
B = 400


def _normalize_body(z_ref, zn_ref):
    z = z_ref[...]
    norm = jnp.sqrt(jnp.sum(z * z, axis=1, keepdims=True))
    zn_ref[...] = (z / jnp.maximum(norm, 1e-12)).astype(jnp.bfloat16)


def _sc_body(zn_bf_hbm, zn_i32_hbm, src_hbm, dst_hbm, out_hbm,
             table, idx_src, idx_dst, s0, d0, s1, d1, scores, accbuf,
             stage_bf, stage_i32, sem0, sem1):
    n = zn_i32_hbm.shape[0]
    per_w = out_hbm.shape[0] // NW
    cid = lax.axis_index("c")
    sid = lax.axis_index("s")
    wid = sid * NC + cid
    base = wid * per_w
    lane = lax.iota(jnp.int32, L)

    sc_rows = stage_bf.shape[0]
    slab = ((n // NS) // 16 + 1) * 16
    start = sid * slab

    @pl.loop(0, slab // sc_rows)
    def _stage(si):
        r0 = start + si * sc_rows

        @pl.when(r0 < n)
        def _do():
            pltpu.sync_copy(zn_bf_hbm.at[pl.ds(r0, sc_rows)], stage_bf)

            @pl.loop(0, sc_rows)
            def _row(r):
                for c in range(D // (2 * L)):
                    v = stage_bf[r, pl.ds(c * 2 * L, 2 * L)]
                    stage_i32[r, pl.ds(c * L, L)] = plsc.bitcast(v, jnp.int32)

            pltpu.sync_copy(stage_i32, table.at[pl.ds(r0, sc_rows)])
    plsc.subcore_barrier()

    def issue(ci, sbuf, dbuf, sem):
        o = ci * C
        pltpu.async_copy(table.at[idx_src.at[pl.ds(o, C)]], sbuf, sem)
        pltpu.async_copy(table.at[idx_dst.at[pl.ds(o, C)]], dbuf, sem)

    def wait(ci_buf, sem):
        sbuf, dbuf = ci_buf
        pltpu.make_async_copy(zn_i32_hbm.at[pl.ds(0, C)], sbuf, sem).wait()
        pltpu.make_async_copy(zn_i32_hbm.at[pl.ds(0, C)], dbuf, sem).wait()

    def compute(ci, sbuf, dbuf):
        @pl.loop(0, C // L)
        def _group(g):
            for j in range(L):
                e = g * L + j
                acc = None
                for c in range(D // (2 * L)):
                    a = plsc.bitcast(sbuf[e, pl.ds(c * L, L)], jnp.bfloat16)
                    b = plsc.bitcast(dbuf[e, pl.ds(c * L, L)], jnp.bfloat16)
                    p = a * b
                    acc = p if acc is None else acc + p
                lo, hi = plsc.unpack(acc, format=plsc.PackFormat.INTERLEAVED)
                accbuf[j, pl.ds(0, L)] = lo + hi
            cols = [
                plsc.load_gather(
                    accbuf, [lane, jnp.full((L,), c, jnp.int32)])
                for c in range(L)
            ]
            while len(cols) > 1:
                cols = [a + b for a, b in zip(cols[::2], cols[1::2])]
            scores[pl.ds(ci * C + g * L, L)] = jnp.exp(2.0 * cols[0] - 2.0)

    nch = B // C

    @pl.loop(0, per_w // B)
    def _block(blk):
        bbase = base + blk * B
        pltpu.sync_copy(src_hbm.at[pl.ds(bbase, B)], idx_src)
        pltpu.sync_copy(dst_hbm.at[pl.ds(bbase, B)], idx_dst)

        issue(0, s0, d0, sem0)

        @pl.loop(0, nch // 2)
        def _pair(it):
            i = 2 * it
            issue(i + 1, s1, d1, sem1)
            wait((s0, d0), sem0)
            compute(i, s0, d0)
            issue(i + 2, s0, d0, sem0)
            wait((s1, d1), sem1)
            compute(i + 1, s1, d1)

        wait((s0, d0), sem0)
        compute(nch - 1, s0, d0)

        pltpu.sync_copy(scores, out_hbm.at[pl.ds(bbase, B)])


def kernel(z, edge_index):
    n, d = z.shape
    e = edge_index.shape[1]
    assert d == D and e % (NW * B) == 0 and (B // C) % 2 == 1 and n % C == 0

    zn = pl.pallas_call(
        _normalize_body,
        out_shape=jax.ShapeDtypeStruct((n, d), jnp.bfloat16),
    )(z)
    zn_i32 = jax.lax.bitcast_convert_type(
        zn.reshape(n, d // 2, 2), jnp.int32)

    idx = edge_index.astype(jnp.int32)
    src = idx[0]
    dst = idx[1]

    cp = pltpu.CompilerParams(use_tc_tiling_on_sc=False)
    if "needs_layout_passes" in pltpu.CompilerParams.__dataclass_fields__:
        cp = dataclasses.replace(cp, needs_layout_passes=False)

    mesh = plsc.VectorSubcoreMesh(core_axis_name="c", subcore_axis_name="s")
    sc_kernel = pl.kernel(
        _sc_body,
        out_type=jax.ShapeDtypeStruct((e,), jnp.float32),
        mesh=mesh,
        scratch_types=[
            pltpu.VMEM_SHARED((n, D // 2), jnp.int32),
            pltpu.VMEM((B,), jnp.int32),
            pltpu.VMEM((B,), jnp.int32),
            pltpu.VMEM((C, D // 2), jnp.int32),
            pltpu.VMEM((C, D // 2), jnp.int32),
            pltpu.VMEM((C, D // 2), jnp.int32),
            pltpu.VMEM((C, D // 2), jnp.int32),
            pltpu.VMEM((B,), jnp.float32),
            pltpu.VMEM((L, L), jnp.float32),
            pltpu.VMEM((C, D), jnp.bfloat16),
            pltpu.VMEM((C, D // 2), jnp.int32),
            pltpu.SemaphoreType.DMA,
            pltpu.SemaphoreType.DMA,
        ],
        compiler_params=cp,
    )
    return sc_kernel(zn, zn_i32, src, dst)
